# Initial kernel scaffold; baseline (speedup 1.0000x reference)
import jax
import jax.numpy as jnp
from jax.experimental import pallas as pl


def kernel(hypergraph, embedding, W, b):
    raise NotImplementedError("write your pallas kernel here")


# speedup vs baseline: 17.7770x; 17.7770x over previous
import functools

import jax
import jax.numpy as jnp
from jax import lax
from jax.experimental import pallas as pl
from jax.experimental.pallas import tpu as pltpu
from jax.experimental.pallas import tpu_sc as plsc

N = 10000
NNZ = 320000
DIM = 128
WIDTH = 144
NW = 32
PAIRS_PER_W = NNZ // NW
CHUNK = 80
CHUNKS_PER_W = PAIRS_PER_W // CHUNK
ROWS_PER_S = N // 16


def _sc_phase_body(table_hbm, src_hbm, dst_hbm, zeros_hbm, out_hbm,
                   src_v, dst_v, rows_v, sem, acc):
    c = lax.axis_index("c")
    s = lax.axis_index("s")
    w = s * 2 + c

    pltpu.sync_copy(zeros_hbm.at[pl.ds(s * ROWS_PER_S, ROWS_PER_S)],
                    acc.at[pl.ds(s * ROWS_PER_S, ROWS_PER_S)])
    pltpu.sync_copy(src_hbm.at[pl.ds(w * CHUNKS_PER_W, CHUNKS_PER_W)], src_v)
    pltpu.sync_copy(dst_hbm.at[pl.ds(w * CHUNKS_PER_W, CHUNKS_PER_W)], dst_v)
    plsc.subcore_barrier()

    @pl.loop(0, CHUNKS_PER_W)
    def _(j):
        pltpu.async_copy(table_hbm.at[src_v.at[j]], rows_v, sem).wait()
        pltpu.sync_copy(rows_v, acc.at[dst_v.at[j]], add=True)

    plsc.subcore_barrier()
    pltpu.sync_copy(acc.at[pl.ds(s * ROWS_PER_S, ROWS_PER_S)],
                    out_hbm.at[c, pl.ds(s * ROWS_PER_S, ROWS_PER_S)])


_sc_phase = pl.kernel(
    _sc_phase_body,
    out_type=jax.ShapeDtypeStruct((2, N, WIDTH), jnp.float32),
    mesh=plsc.VectorSubcoreMesh(core_axis_name="c", subcore_axis_name="s"),
    scratch_types=[
        pltpu.VMEM((CHUNKS_PER_W, CHUNK), jnp.int32),
        pltpu.VMEM((CHUNKS_PER_W, CHUNK), jnp.int32),
        pltpu.VMEM((CHUNK, WIDTH), jnp.float32),
        pltpu.SemaphoreType.DMA,
        pltpu.VMEM_SHARED((N, WIDTH), jnp.float32),
    ],
    compiler_params=pltpu.CompilerParams(use_tc_tiling_on_sc=False),
)


ROWS_BLK = 400


def _tc_mid_body(ep_ref, w_ref, out_ref):
    es = ep_ref[0] + ep_ref[1]
    feat = es[:, :DIM]
    cnt = es[:, DIM]
    inv = jnp.where(cnt > 0, 1.0 / cnt, 0.0)
    y = jnp.dot(feat, w_ref[...], preferred_element_type=jnp.float32)
    y = y * inv[:, None]
    col = lax.broadcasted_iota(jnp.int32, (ROWS_BLK, WIDTH - DIM), 1)
    pad = jnp.where(col == 0, 1.0, 0.0).astype(jnp.float32)
    out_ref[...] = jnp.concatenate([y, pad], axis=1)


def _tc_mid(e_p, W):
    return pl.pallas_call(
        _tc_mid_body,
        grid=(N // ROWS_BLK,),
        in_specs=[
            pl.BlockSpec((2, ROWS_BLK, WIDTH), lambda i: (0, i, 0)),
            pl.BlockSpec((DIM, DIM), lambda i: (0, 0)),
        ],
        out_specs=pl.BlockSpec((ROWS_BLK, WIDTH), lambda i: (i, 0)),
        out_shape=jax.ShapeDtypeStruct((N, WIDTH), jnp.float32),
    )(e_p, W)


def _tc_final_body(op_ref, b_ref, out_ref):
    os_ = op_ref[0] + op_ref[1]
    cnt = os_[:, DIM]
    inv = jnp.where(cnt > 0, 1.0 / cnt, 0.0)
    out_ref[...] = os_[:, :DIM] * inv[:, None] + b_ref[...]


def _tc_final(out_p, b2d):
    return pl.pallas_call(
        _tc_final_body,
        grid=(N // ROWS_BLK,),
        in_specs=[
            pl.BlockSpec((2, ROWS_BLK, WIDTH), lambda i: (0, i, 0)),
            pl.BlockSpec((1, DIM), lambda i: (0, 0)),
        ],
        out_specs=pl.BlockSpec((ROWS_BLK, DIM), lambda i: (i, 0)),
        out_shape=jax.ShapeDtypeStruct((N, DIM), jnp.float32),
    )(out_p, b2d)


@jax.jit
def kernel(hypergraph, embedding, W, b):
    node_idx = hypergraph[0].reshape(NNZ // CHUNK, CHUNK)
    edge_idx = hypergraph[1].reshape(NNZ // CHUNK, CHUNK)
    onescol = jnp.concatenate(
        [jnp.ones((N, 1), jnp.float32), jnp.zeros((N, WIDTH - DIM - 1), jnp.float32)],
        axis=1)
    xa = jnp.concatenate([embedding, onescol], axis=1)
    zeros = jnp.zeros((N, WIDTH), jnp.float32)

    e_p = _sc_phase(xa, node_idx, edge_idx, zeros)
    ea = _tc_mid(e_p, W)
    out_p = _sc_phase(ea, edge_idx, node_idx, zeros)
    return _tc_final(out_p, b.reshape(1, DIM))


# speedup vs baseline: 21.7137x; 1.2214x over previous
import jax
import jax.numpy as jnp
from jax import lax
from jax.experimental import pallas as pl
from jax.experimental.pallas import tpu as pltpu
from jax.experimental.pallas import tpu_sc as plsc

N = 10000
NNZ = 320000
DIM = 128
WIDTH = 144
NW = 32
PAIRS_PER_W = NNZ // NW
CHUNK = 50
CHUNKS_PER_W = PAIRS_PER_W // CHUNK
ROWS_PER_S = N // 16


def _sc_phase_body(table_hbm, src_hbm, dst_hbm, zeros_hbm, out_hbm,
                   src_v, dst_v, buf0, buf1, sem0, sem1, acc):
    c = lax.axis_index("c")
    s = lax.axis_index("s")
    w = s * 2 + c

    pltpu.sync_copy(src_hbm.at[pl.ds(w * CHUNKS_PER_W, CHUNKS_PER_W)], src_v)
    pltpu.sync_copy(dst_hbm.at[pl.ds(w * CHUNKS_PER_W, CHUNKS_PER_W)], dst_v)
    pltpu.async_copy(table_hbm.at[src_v.at[0]], buf0, sem0)
    pltpu.sync_copy(zeros_hbm.at[pl.ds(s * ROWS_PER_S, ROWS_PER_S)],
                    acc.at[pl.ds(s * ROWS_PER_S, ROWS_PER_S)])
    plsc.subcore_barrier()

    @pl.loop(0, CHUNKS_PER_W - 2, step=2)
    def _(j):
        pltpu.async_copy(table_hbm.at[src_v.at[j + 1]], buf1, sem1)
        pltpu.make_async_copy(table_hbm.at[src_v.at[j]], buf0, sem0).wait()
        pltpu.sync_copy(buf0, acc.at[dst_v.at[j]], add=True)
        pltpu.async_copy(table_hbm.at[src_v.at[j + 2]], buf0, sem0)
        pltpu.make_async_copy(table_hbm.at[src_v.at[j + 1]], buf1, sem1).wait()
        pltpu.sync_copy(buf1, acc.at[dst_v.at[j + 1]], add=True)

    last = CHUNKS_PER_W - 1
    pltpu.async_copy(table_hbm.at[src_v.at[last]], buf1, sem1)
    pltpu.make_async_copy(table_hbm.at[src_v.at[last - 1]], buf0, sem0).wait()
    pltpu.sync_copy(buf0, acc.at[dst_v.at[last - 1]], add=True)
    pltpu.make_async_copy(table_hbm.at[src_v.at[last]], buf1, sem1).wait()
    pltpu.sync_copy(buf1, acc.at[dst_v.at[last]], add=True)

    plsc.subcore_barrier()
    pltpu.sync_copy(acc.at[pl.ds(s * ROWS_PER_S, ROWS_PER_S)],
                    out_hbm.at[c, pl.ds(s * ROWS_PER_S, ROWS_PER_S)])


_sc_phase = pl.kernel(
    _sc_phase_body,
    out_type=jax.ShapeDtypeStruct((2, N, WIDTH), jnp.float32),
    mesh=plsc.VectorSubcoreMesh(core_axis_name="c", subcore_axis_name="s"),
    scratch_types=[
        pltpu.VMEM((CHUNKS_PER_W, CHUNK), jnp.int32),
        pltpu.VMEM((CHUNKS_PER_W, CHUNK), jnp.int32),
        pltpu.VMEM((CHUNK, WIDTH), jnp.float32),
        pltpu.VMEM((CHUNK, WIDTH), jnp.float32),
        pltpu.SemaphoreType.DMA,
        pltpu.SemaphoreType.DMA,
        pltpu.VMEM_SHARED((N, WIDTH), jnp.float32),
    ],
    compiler_params=pltpu.CompilerParams(use_tc_tiling_on_sc=False),
)


ROWS_BLK = 400


def _aug_pad(y):
    col = lax.broadcasted_iota(jnp.int32, (ROWS_BLK, WIDTH - DIM), 1)
    pad = jnp.where(col == 0, 1.0, 0.0).astype(jnp.float32)
    return jnp.concatenate([y, pad], axis=1)


def _tc_pre_body(e_ref, w_ref, out_ref):
    y = jnp.dot(e_ref[...], w_ref[...], preferred_element_type=jnp.float32)
    out_ref[...] = _aug_pad(y)


def _tc_pre(emb, W):
    return pl.pallas_call(
        _tc_pre_body,
        grid=(N // ROWS_BLK,),
        in_specs=[
            pl.BlockSpec((ROWS_BLK, DIM), lambda i: (i, 0)),
            pl.BlockSpec((DIM, DIM), lambda i: (0, 0)),
        ],
        out_specs=pl.BlockSpec((ROWS_BLK, WIDTH), lambda i: (i, 0)),
        out_shape=jax.ShapeDtypeStruct((N, WIDTH), jnp.float32),
    )(emb, W)


def _tc_mid_body(ep_ref, out_ref):
    es = ep_ref[0] + ep_ref[1]
    cnt = es[:, DIM]
    inv = jnp.where(cnt > 0, 1.0 / cnt, 0.0)
    out_ref[...] = _aug_pad(es[:, :DIM] * inv[:, None])


def _tc_mid(e_p):
    return pl.pallas_call(
        _tc_mid_body,
        grid=(N // ROWS_BLK,),
        in_specs=[
            pl.BlockSpec((2, ROWS_BLK, WIDTH), lambda i: (0, i, 0)),
        ],
        out_specs=pl.BlockSpec((ROWS_BLK, WIDTH), lambda i: (i, 0)),
        out_shape=jax.ShapeDtypeStruct((N, WIDTH), jnp.float32),
    )(e_p)


def _tc_final_body(op_ref, b_ref, out_ref):
    os_ = op_ref[0] + op_ref[1]
    cnt = os_[:, DIM]
    inv = jnp.where(cnt > 0, 1.0 / cnt, 0.0)
    out_ref[...] = os_[:, :DIM] * inv[:, None] + b_ref[...]


def _tc_final(out_p, b2d):
    return pl.pallas_call(
        _tc_final_body,
        grid=(N // ROWS_BLK,),
        in_specs=[
            pl.BlockSpec((2, ROWS_BLK, WIDTH), lambda i: (0, i, 0)),
            pl.BlockSpec((1, DIM), lambda i: (0, 0)),
        ],
        out_specs=pl.BlockSpec((ROWS_BLK, DIM), lambda i: (i, 0)),
        out_shape=jax.ShapeDtypeStruct((N, DIM), jnp.float32),
    )(out_p, b2d)


@jax.jit
def kernel(hypergraph, embedding, W, b):
    node_idx = hypergraph[0].reshape(NNZ // CHUNK, CHUNK)
    edge_idx = hypergraph[1].reshape(NNZ // CHUNK, CHUNK)
    zeros = jnp.zeros((N, WIDTH), jnp.float32)

    xa = _tc_pre(embedding, W)
    e_p = _sc_phase(xa, node_idx, edge_idx, zeros)
    ea = _tc_mid(e_p)
    out_p = _sc_phase(ea, edge_idx, node_idx, zeros)
    return _tc_final(out_p, b.reshape(1, DIM))


# speedup vs baseline: 27.3085x; 1.2577x over previous
import jax
import jax.numpy as jnp
from jax import lax
from jax.experimental import pallas as pl
from jax.experimental.pallas import tpu as pltpu
from jax.experimental.pallas import tpu_sc as plsc

N = 10000
NNZ = 320000
DIM = 128
WIDTH = 144
NW = 32
PAIRS_PER_W = NNZ // NW
CHUNK = 80
CHUNKS_PER_W = PAIRS_PER_W // CHUNK
GROUPS = 5
GCHUNKS = CHUNKS_PER_W // GROUPS
GPAIRS = GCHUNKS * CHUNK
ROWS_PER_S = N // 16


def _make_sc_phase(src_row, dst_row):
    def body(table_hbm, hyper_hbm, zeros_hbm, out_hbm,
             src_a, dst_a, src_b, dst_b, buf0, buf1, sem0, sem1, gsem, acc):
        c = lax.axis_index("c")
        s = lax.axis_index("s")
        w = s * 2 + c
        base = w * PAIRS_PER_W

        def stage(grp, sv, dv):
            pltpu.async_copy(
                hyper_hbm.at[src_row, pl.ds(base + grp * GPAIRS, GPAIRS)], sv, gsem)
            pltpu.async_copy(
                hyper_hbm.at[dst_row, pl.ds(base + grp * GPAIRS, GPAIRS)], dv, gsem)

        def stage_wait(grp, sv, dv):
            pltpu.make_async_copy(
                hyper_hbm.at[src_row, pl.ds(base + grp * GPAIRS, GPAIRS)], sv,
                gsem).wait()
            pltpu.make_async_copy(
                hyper_hbm.at[dst_row, pl.ds(base + grp * GPAIRS, GPAIRS)], dv,
                gsem).wait()

        stage(0, src_a, dst_a)
        pltpu.sync_copy(zeros_hbm.at[pl.ds(s * ROWS_PER_S, ROWS_PER_S)],
                        acc.at[pl.ds(s * ROWS_PER_S, ROWS_PER_S)])
        stage_wait(0, src_a, dst_a)
        pltpu.async_copy(table_hbm.at[src_a.at[pl.ds(0, CHUNK)]], buf0, sem0)
        plsc.subcore_barrier()

        for grp in range(GROUPS):
            sv, dv = (src_a, dst_a) if grp % 2 == 0 else (src_b, dst_b)
            nsv, ndv = (src_b, dst_b) if grp % 2 == 0 else (src_a, dst_a)
            if grp + 1 < GROUPS:
                stage(grp + 1, nsv, ndv)

            def g(j, sv=sv):
                return sv.at[pl.ds(j * CHUNK, CHUNK)]

            def sc(j, dv=dv):
                return acc.at[dv.at[pl.ds(j * CHUNK, CHUNK)]]

            @pl.loop(0, GCHUNKS - 2, step=2)
            def _(j, g=g, sc=sc):
                pltpu.async_copy(table_hbm.at[g(j + 1)], buf1, sem1)
                pltpu.make_async_copy(table_hbm.at[g(j)], buf0, sem0).wait()
                pltpu.sync_copy(buf0, sc(j), add=True)
                pltpu.async_copy(table_hbm.at[g(j + 2)], buf0, sem0)
                pltpu.make_async_copy(table_hbm.at[g(j + 1)], buf1, sem1).wait()
                pltpu.sync_copy(buf1, sc(j + 1), add=True)

            last = GCHUNKS - 1
            pltpu.make_async_copy(table_hbm.at[g(last)], buf0, sem0).wait()
            pltpu.sync_copy(buf0, sc(last), add=True)
            if grp + 1 < GROUPS:
                stage_wait(grp + 1, nsv, ndv)
                pltpu.async_copy(table_hbm.at[nsv.at[pl.ds(0, CHUNK)]], buf0, sem0)

        plsc.subcore_barrier()
        pltpu.sync_copy(acc.at[pl.ds(s * ROWS_PER_S, ROWS_PER_S)],
                        out_hbm.at[c, pl.ds(s * ROWS_PER_S, ROWS_PER_S)])

    return pl.kernel(
        body,
        out_type=jax.ShapeDtypeStruct((2, N, WIDTH), jnp.float32),
        mesh=plsc.VectorSubcoreMesh(core_axis_name="c", subcore_axis_name="s"),
        scratch_types=[
            pltpu.VMEM((GPAIRS,), jnp.int32),
            pltpu.VMEM((GPAIRS,), jnp.int32),
            pltpu.VMEM((GPAIRS,), jnp.int32),
            pltpu.VMEM((GPAIRS,), jnp.int32),
            pltpu.VMEM((CHUNK, WIDTH), jnp.float32),
            pltpu.VMEM((CHUNK, WIDTH), jnp.float32),
            pltpu.SemaphoreType.DMA,
            pltpu.SemaphoreType.DMA,
            pltpu.SemaphoreType.DMA,
            pltpu.VMEM_SHARED((N, WIDTH), jnp.float32),
        ],
        compiler_params=pltpu.CompilerParams(use_tc_tiling_on_sc=False),
    )


_sc_phase1 = _make_sc_phase(0, 1)
_sc_phase2 = _make_sc_phase(1, 0)


ROWS_BLK = 2000


def _aug_pad(y):
    col = lax.broadcasted_iota(jnp.int32, (ROWS_BLK, WIDTH - DIM), 1)
    pad = jnp.where(col == 0, 1.0, 0.0).astype(jnp.float32)
    return jnp.concatenate([y, pad], axis=1)


def _tc_pre_body(e_ref, w_ref, out_ref):
    y = jnp.dot(e_ref[...], w_ref[...], preferred_element_type=jnp.float32)
    out_ref[...] = _aug_pad(y)


def _tc_pre(emb, W):
    return pl.pallas_call(
        _tc_pre_body,
        grid=(N // ROWS_BLK,),
        in_specs=[
            pl.BlockSpec((ROWS_BLK, DIM), lambda i: (i, 0)),
            pl.BlockSpec((DIM, DIM), lambda i: (0, 0)),
        ],
        out_specs=pl.BlockSpec((ROWS_BLK, WIDTH), lambda i: (i, 0)),
        out_shape=jax.ShapeDtypeStruct((N, WIDTH), jnp.float32),
    )(emb, W)


def _tc_mid_body(ep_ref, out_ref):
    es = ep_ref[0] + ep_ref[1]
    cnt = es[:, DIM]
    inv = jnp.where(cnt > 0, 1.0 / cnt, 0.0)
    out_ref[...] = _aug_pad(es[:, :DIM] * inv[:, None])


def _tc_mid(e_p):
    return pl.pallas_call(
        _tc_mid_body,
        grid=(N // ROWS_BLK,),
        in_specs=[
            pl.BlockSpec((2, ROWS_BLK, WIDTH), lambda i: (0, i, 0)),
        ],
        out_specs=pl.BlockSpec((ROWS_BLK, WIDTH), lambda i: (i, 0)),
        out_shape=jax.ShapeDtypeStruct((N, WIDTH), jnp.float32),
    )(e_p)


def _tc_final_body(op_ref, b_ref, out_ref):
    os_ = op_ref[0] + op_ref[1]
    cnt = os_[:, DIM]
    inv = jnp.where(cnt > 0, 1.0 / cnt, 0.0)
    out_ref[...] = os_[:, :DIM] * inv[:, None] + b_ref[...]


def _tc_final(out_p, b2d):
    return pl.pallas_call(
        _tc_final_body,
        grid=(N // ROWS_BLK,),
        in_specs=[
            pl.BlockSpec((2, ROWS_BLK, WIDTH), lambda i: (0, i, 0)),
            pl.BlockSpec((1, DIM), lambda i: (0, 0)),
        ],
        out_specs=pl.BlockSpec((ROWS_BLK, DIM), lambda i: (i, 0)),
        out_shape=jax.ShapeDtypeStruct((N, DIM), jnp.float32),
    )(out_p, b2d)


@jax.jit
def kernel(hypergraph, embedding, W, b):
    zeros = jnp.zeros((N, WIDTH), jnp.float32)

    xa = _tc_pre(embedding, W)
    e_p = _sc_phase1(xa, hypergraph, zeros)
    ea = _tc_mid(e_p)
    out_p = _sc_phase2(ea, hypergraph, zeros)
    return _tc_final(out_p, b.reshape(1, DIM))


# speedup vs baseline: 35.4702x; 1.2989x over previous
import jax
import jax.numpy as jnp
from jax import lax
from jax.experimental import pallas as pl
from jax.experimental.pallas import tpu as pltpu
from jax.experimental.pallas import tpu_sc as plsc

N = 10000
NNZ = 320000
DIM = 128
NW = 32
PAIRS_PER_W = NNZ // NW
CHUNK = 80
CHUNKS_PER_W = PAIRS_PER_W // CHUNK
GROUPS = 5
GCHUNKS = CHUNKS_PER_W // GROUPS
GPAIRS = GCHUNKS * CHUNK
ROWS_PER_S = N // 16
LANES = 16
ROWS_BLK = 2000


def _make_sc_phase(src_row, dst_row, with_hist):
    def body(*refs):
        if with_hist:
            (table_hbm, hyper_hbm, zeros_hbm,
             out_hbm, histd_hbm, histb_hbm,
             src_a, dst_a, src_b, dst_b, buf0, buf1,
             hist_d, hist_b, sem0, sem1, gsem, acc) = refs
        else:
            (table_hbm, hyper_hbm, zeros_hbm, out_hbm,
             src_a, dst_a, src_b, dst_b, buf0, buf1,
             sem0, sem1, gsem, acc) = refs
        c = lax.axis_index("c")
        s = lax.axis_index("s")
        w = s * 2 + c
        base = w * PAIRS_PER_W

        def stage(grp, sv, dv):
            pltpu.async_copy(
                hyper_hbm.at[src_row, pl.ds(base + grp * GPAIRS, GPAIRS)], sv, gsem)
            pltpu.async_copy(
                hyper_hbm.at[dst_row, pl.ds(base + grp * GPAIRS, GPAIRS)], dv, gsem)

        def stage_wait(grp, sv, dv):
            pltpu.make_async_copy(
                hyper_hbm.at[src_row, pl.ds(base + grp * GPAIRS, GPAIRS)], sv,
                gsem).wait()
            pltpu.make_async_copy(
                hyper_hbm.at[dst_row, pl.ds(base + grp * GPAIRS, GPAIRS)], dv,
                gsem).wait()

        stage(0, src_a, dst_a)
        pltpu.sync_copy(zeros_hbm.at[pl.ds(s * ROWS_PER_S, ROWS_PER_S)],
                        acc.at[pl.ds(s * ROWS_PER_S, ROWS_PER_S)])
        if with_hist:
            zv = jnp.zeros((LANES,), jnp.float32)

            @pl.loop(0, N // LANES)
            def _(i):
                hist_d[pl.ds(i * LANES, LANES)] = zv
                hist_b[pl.ds(i * LANES, LANES)] = zv

        stage_wait(0, src_a, dst_a)
        pltpu.async_copy(table_hbm.at[src_a.at[pl.ds(0, CHUNK)]], buf0, sem0)
        plsc.subcore_barrier()

        ones = jnp.ones((LANES,), jnp.float32)

        def hist(j, sv, dv):
            if not with_hist:
                return
            for q in range(CHUNK // LANES):
                idx_s = sv[pl.ds(j * CHUNK + q * LANES, LANES)]
                plsc.addupdate_scatter(hist_d, [idx_s], ones)
                idx_d = dv[pl.ds(j * CHUNK + q * LANES, LANES)]
                plsc.addupdate_scatter(hist_b, [idx_d], ones)

        for grp in range(GROUPS):
            sv, dv = (src_a, dst_a) if grp % 2 == 0 else (src_b, dst_b)
            nsv, ndv = (src_b, dst_b) if grp % 2 == 0 else (src_a, dst_a)
            if grp + 1 < GROUPS:
                stage(grp + 1, nsv, ndv)

            def g(j, sv=sv):
                return sv.at[pl.ds(j * CHUNK, CHUNK)]

            def sc(j, dv=dv):
                return acc.at[dv.at[pl.ds(j * CHUNK, CHUNK)]]

            @pl.loop(0, GCHUNKS - 2, step=2)
            def _(j, g=g, sc=sc, sv=sv, dv=dv):
                pltpu.async_copy(table_hbm.at[g(j + 1)], buf1, sem1)
                hist(j, sv, dv)
                pltpu.make_async_copy(table_hbm.at[g(j)], buf0, sem0).wait()
                pltpu.sync_copy(buf0, sc(j), add=True)
                pltpu.async_copy(table_hbm.at[g(j + 2)], buf0, sem0)
                hist(j + 1, sv, dv)
                pltpu.make_async_copy(table_hbm.at[g(j + 1)], buf1, sem1).wait()
                pltpu.sync_copy(buf1, sc(j + 1), add=True)

            last = GCHUNKS - 1
            hist(last, sv, dv)
            pltpu.make_async_copy(table_hbm.at[g(last)], buf0, sem0).wait()
            pltpu.sync_copy(buf0, sc(last), add=True)
            if grp + 1 < GROUPS:
                stage_wait(grp + 1, nsv, ndv)
                pltpu.async_copy(table_hbm.at[nsv.at[pl.ds(0, CHUNK)]], buf0, sem0)

        plsc.subcore_barrier()
        pltpu.sync_copy(acc.at[pl.ds(s * ROWS_PER_S, ROWS_PER_S)],
                        out_hbm.at[c, pl.ds(s * ROWS_PER_S, ROWS_PER_S)])
        if with_hist:
            for i in range(N // ROWS_BLK):
                pltpu.sync_copy(hist_d.at[pl.ds(i * ROWS_BLK, ROWS_BLK)],
                                histd_hbm.at[i, w])
                pltpu.sync_copy(hist_b.at[pl.ds(i * ROWS_BLK, ROWS_BLK)],
                                histb_hbm.at[i, w])

    out_type = [jax.ShapeDtypeStruct((2, N, DIM), jnp.float32)]
    scratch = [
        pltpu.VMEM((GPAIRS,), jnp.int32),
        pltpu.VMEM((GPAIRS,), jnp.int32),
        pltpu.VMEM((GPAIRS,), jnp.int32),
        pltpu.VMEM((GPAIRS,), jnp.int32),
        pltpu.VMEM((CHUNK, DIM), jnp.float32),
        pltpu.VMEM((CHUNK, DIM), jnp.float32),
    ]
    if with_hist:
        out_type += [jax.ShapeDtypeStruct((N // ROWS_BLK, NW, ROWS_BLK),
                                          jnp.float32),
                     jax.ShapeDtypeStruct((N // ROWS_BLK, NW, ROWS_BLK),
                                          jnp.float32)]
        scratch += [pltpu.VMEM((N,), jnp.float32), pltpu.VMEM((N,), jnp.float32)]
    scratch += [
        pltpu.SemaphoreType.DMA,
        pltpu.SemaphoreType.DMA,
        pltpu.SemaphoreType.DMA,
        pltpu.VMEM_SHARED((N, DIM), jnp.float32),
    ]
    return pl.kernel(
        body,
        out_type=tuple(out_type) if with_hist else out_type[0],
        mesh=plsc.VectorSubcoreMesh(core_axis_name="c", subcore_axis_name="s"),
        scratch_types=scratch,
        compiler_params=pltpu.CompilerParams(use_tc_tiling_on_sc=False,
                                             needs_layout_passes=False),
    )


_sc_phase1 = _make_sc_phase(0, 1, True)
_sc_phase2 = _make_sc_phase(1, 0, False)


def _tc_pre_body(e_ref, w_ref, out_ref):
    out_ref[...] = jnp.dot(e_ref[...], w_ref[...],
                           preferred_element_type=jnp.float32)


def _tc_pre(emb, W):
    return pl.pallas_call(
        _tc_pre_body,
        grid=(N // ROWS_BLK,),
        in_specs=[
            pl.BlockSpec((ROWS_BLK, DIM), lambda i: (i, 0)),
            pl.BlockSpec((DIM, DIM), lambda i: (0, 0)),
        ],
        out_specs=pl.BlockSpec((ROWS_BLK, DIM), lambda i: (i, 0)),
        out_shape=jax.ShapeDtypeStruct((N, DIM), jnp.float32),
    )(emb, W)


def _inv_seg(h_ref):
    seg = jnp.sum(h_ref[0], axis=0)
    return jnp.where(seg > 0, 1.0 / seg, 0.0)


def _tc_mid_body(ep_ref, hb_ref, out_ref):
    out_ref[...] = (ep_ref[0] + ep_ref[1]) * _inv_seg(hb_ref)[:, None]


def _tc_mid(e_p, histb):
    return pl.pallas_call(
        _tc_mid_body,
        grid=(N // ROWS_BLK,),
        in_specs=[
            pl.BlockSpec((2, ROWS_BLK, DIM), lambda i: (0, i, 0)),
            pl.BlockSpec((1, NW, ROWS_BLK), lambda i: (i, 0, 0)),
        ],
        out_specs=pl.BlockSpec((ROWS_BLK, DIM), lambda i: (i, 0)),
        out_shape=jax.ShapeDtypeStruct((N, DIM), jnp.float32),
    )(e_p, histb)


def _tc_final_body(op_ref, hd_ref, b_ref, out_ref):
    out_ref[...] = ((op_ref[0] + op_ref[1]) * _inv_seg(hd_ref)[:, None]
                    + b_ref[...])


def _tc_final(out_p, histd, b2d):
    return pl.pallas_call(
        _tc_final_body,
        grid=(N // ROWS_BLK,),
        in_specs=[
            pl.BlockSpec((2, ROWS_BLK, DIM), lambda i: (0, i, 0)),
            pl.BlockSpec((1, NW, ROWS_BLK), lambda i: (i, 0, 0)),
            pl.BlockSpec((1, DIM), lambda i: (0, 0)),
        ],
        out_specs=pl.BlockSpec((ROWS_BLK, DIM), lambda i: (i, 0)),
        out_shape=jax.ShapeDtypeStruct((N, DIM), jnp.float32),
    )(out_p, histd, b2d)


@jax.jit
def kernel(hypergraph, embedding, W, b):
    zeros = jnp.zeros((N, DIM), jnp.float32)

    xa = _tc_pre(embedding, W)
    e_p, histd, histb = _sc_phase1(xa, hypergraph, zeros)
    ea = _tc_mid(e_p, histb)
    out_p = _sc_phase2(ea, hypergraph, zeros)
    return _tc_final(out_p, histd, b.reshape(1, DIM))


# speedup vs baseline: 38.9482x; 1.0981x over previous
import jax
import jax.numpy as jnp
from jax import lax
from jax.experimental import pallas as pl
from jax.experimental.pallas import tpu as pltpu
from jax.experimental.pallas import tpu_sc as plsc

N = 10000
NNZ = 320000
DIM = 128
NW = 32
PAIRS_PER_W = NNZ // NW
CHUNK = 80
CHUNKS_PER_W = PAIRS_PER_W // CHUNK
GROUPS = 5
GCHUNKS = CHUNKS_PER_W // GROUPS
GPAIRS = GCHUNKS * CHUNK
ROWS_PER_S = N // 16
LANES = 16
ROWS_BLK = 2000


def _make_sc_phase(src_row, dst_row):
    def body(table_hbm, hyper_hbm, zeros_hbm, out_hbm, hist_hbm,
             src_a, dst_a, src_b, dst_b, b0, b1, b2, hist,
             sg0, sg1, sg2, ss0, ss1, ss2, gsem, acc):
        c = lax.axis_index("c")
        s = lax.axis_index("s")
        w = s * 2 + c
        base = w * PAIRS_PER_W
        bufs = (b0, b1, b2)
        gsems = (sg0, sg1, sg2)
        ssems = (ss0, ss1, ss2)

        def stage(grp, sv, dv):
            pltpu.async_copy(
                hyper_hbm.at[src_row, pl.ds(base + grp * GPAIRS, GPAIRS)], sv, gsem)
            pltpu.async_copy(
                hyper_hbm.at[dst_row, pl.ds(base + grp * GPAIRS, GPAIRS)], dv, gsem)

        def stage_wait(grp, sv, dv):
            pltpu.make_async_copy(
                hyper_hbm.at[src_row, pl.ds(base + grp * GPAIRS, GPAIRS)], sv,
                gsem).wait()
            pltpu.make_async_copy(
                hyper_hbm.at[dst_row, pl.ds(base + grp * GPAIRS, GPAIRS)], dv,
                gsem).wait()

        def ig(ch, p, sv):
            pltpu.async_copy(table_hbm.at[sv.at[pl.ds(ch * CHUNK, CHUNK)]],
                             bufs[p], gsems[p])

        def wg(ch, p, sv):
            pltpu.make_async_copy(table_hbm.at[sv.at[pl.ds(ch * CHUNK, CHUNK)]],
                                  bufs[p], gsems[p]).wait()

        def asc(ch, p, dv):
            pltpu.async_copy(bufs[p], acc.at[dv.at[pl.ds(ch * CHUNK, CHUNK)]],
                             ssems[p], add=True)

        def wsc(ch, p, dv):
            pltpu.make_async_copy(bufs[p], acc.at[dv.at[pl.ds(ch * CHUNK, CHUNK)]],
                                  ssems[p]).wait()

        ones = jnp.ones((LANES,), jnp.float32)

        def hst(ch, dv):
            for q in range(CHUNK // LANES):
                idx = dv[pl.ds(ch * CHUNK + q * LANES, LANES)]
                plsc.addupdate_scatter(hist, [idx], ones)

        stage(0, src_a, dst_a)
        pltpu.sync_copy(zeros_hbm.at[pl.ds(s * ROWS_PER_S, ROWS_PER_S)],
                        acc.at[pl.ds(s * ROWS_PER_S, ROWS_PER_S)])
        zv = jnp.zeros((LANES,), jnp.float32)

        @pl.loop(0, N // LANES)
        def _(i):
            hist[pl.ds(i * LANES, LANES)] = zv

        stage_wait(0, src_a, dst_a)
        ig(0, 0, src_a)
        ig(1, 1, src_a)
        plsc.subcore_barrier()

        for grp in range(GROUPS):
            sv, dv = (src_a, dst_a) if grp % 2 == 0 else (src_b, dst_b)
            nsv, ndv = (src_b, dst_b) if grp % 2 == 0 else (src_a, dst_a)
            if grp + 1 < GROUPS:
                stage(grp + 1, nsv, ndv)

            wg(0, 0, sv)
            asc(0, 0, dv)
            hst(0, dv)
            ig(2, 2, sv)

            @pl.loop(1, GCHUNKS - 2, step=3)
            def _(j, sv=sv, dv=dv):
                wg(j, 1, sv)
                asc(j, 1, dv)
                hst(j, dv)
                wsc(j - 1, 0, dv)
                ig(j + 2, 0, sv)
                wg(j + 1, 2, sv)
                asc(j + 1, 2, dv)
                hst(j + 1, dv)
                wsc(j, 1, dv)

                @pl.when(j <= GCHUNKS - 4)
                def _():
                    ig(j + 3, 1, sv)

                wg(j + 2, 0, sv)
                asc(j + 2, 0, dv)
                hst(j + 2, dv)
                wsc(j + 1, 2, dv)

                @pl.when(j <= GCHUNKS - 5)
                def _():
                    ig(j + 4, 2, sv)

            wsc(GCHUNKS - 1, 0, dv)
            if grp + 1 < GROUPS:
                stage_wait(grp + 1, nsv, ndv)
                ig(0, 0, nsv)
                ig(1, 1, nsv)

        plsc.subcore_barrier()
        pltpu.sync_copy(acc.at[pl.ds(s * ROWS_PER_S, ROWS_PER_S)],
                        out_hbm.at[c, pl.ds(s * ROWS_PER_S, ROWS_PER_S)])
        for i in range(N // ROWS_BLK):
            pltpu.sync_copy(hist.at[pl.ds(i * ROWS_BLK, ROWS_BLK)],
                            hist_hbm.at[i, w])

    return pl.kernel(
        body,
        out_type=(jax.ShapeDtypeStruct((2, N, DIM), jnp.float32),
                  jax.ShapeDtypeStruct((N // ROWS_BLK, NW, ROWS_BLK),
                                       jnp.float32)),
        mesh=plsc.VectorSubcoreMesh(core_axis_name="c", subcore_axis_name="s"),
        scratch_types=[
            pltpu.VMEM((GPAIRS,), jnp.int32),
            pltpu.VMEM((GPAIRS,), jnp.int32),
            pltpu.VMEM((GPAIRS,), jnp.int32),
            pltpu.VMEM((GPAIRS,), jnp.int32),
            pltpu.VMEM((CHUNK, DIM), jnp.float32),
            pltpu.VMEM((CHUNK, DIM), jnp.float32),
            pltpu.VMEM((CHUNK, DIM), jnp.float32),
            pltpu.VMEM((N,), jnp.float32),
            pltpu.SemaphoreType.DMA,
            pltpu.SemaphoreType.DMA,
            pltpu.SemaphoreType.DMA,
            pltpu.SemaphoreType.DMA,
            pltpu.SemaphoreType.DMA,
            pltpu.SemaphoreType.DMA,
            pltpu.SemaphoreType.DMA,
            pltpu.VMEM_SHARED((N, DIM), jnp.float32),
        ],
        compiler_params=pltpu.CompilerParams(use_tc_tiling_on_sc=False,
                                             needs_layout_passes=False),
    )


_sc_phase1 = _make_sc_phase(0, 1)
_sc_phase2 = _make_sc_phase(1, 0)


def _tc_pre_body(e_ref, w_ref, out_ref):
    out_ref[...] = jnp.dot(e_ref[...], w_ref[...],
                           preferred_element_type=jnp.float32)


def _tc_pre(emb, W):
    return pl.pallas_call(
        _tc_pre_body,
        grid=(N // ROWS_BLK,),
        in_specs=[
            pl.BlockSpec((ROWS_BLK, DIM), lambda i: (i, 0)),
            pl.BlockSpec((DIM, DIM), lambda i: (0, 0)),
        ],
        out_specs=pl.BlockSpec((ROWS_BLK, DIM), lambda i: (i, 0)),
        out_shape=jax.ShapeDtypeStruct((N, DIM), jnp.float32),
    )(emb, W)


def _inv_seg(h_ref):
    seg = jnp.sum(h_ref[0], axis=0)
    return jnp.where(seg > 0, 1.0 / seg, 0.0)


def _tc_mid_body(ep_ref, hb_ref, out_ref):
    out_ref[...] = (ep_ref[0] + ep_ref[1]) * _inv_seg(hb_ref)[:, None]


def _tc_mid(e_p, histb):
    return pl.pallas_call(
        _tc_mid_body,
        grid=(N // ROWS_BLK,),
        in_specs=[
            pl.BlockSpec((2, ROWS_BLK, DIM), lambda i: (0, i, 0)),
            pl.BlockSpec((1, NW, ROWS_BLK), lambda i: (i, 0, 0)),
        ],
        out_specs=pl.BlockSpec((ROWS_BLK, DIM), lambda i: (i, 0)),
        out_shape=jax.ShapeDtypeStruct((N, DIM), jnp.float32),
    )(e_p, histb)


def _tc_final_body(op_ref, hd_ref, b_ref, out_ref):
    out_ref[...] = ((op_ref[0] + op_ref[1]) * _inv_seg(hd_ref)[:, None]
                    + b_ref[...])


def _tc_final(out_p, histd, b2d):
    return pl.pallas_call(
        _tc_final_body,
        grid=(N // ROWS_BLK,),
        in_specs=[
            pl.BlockSpec((2, ROWS_BLK, DIM), lambda i: (0, i, 0)),
            pl.BlockSpec((1, NW, ROWS_BLK), lambda i: (i, 0, 0)),
            pl.BlockSpec((1, DIM), lambda i: (0, 0)),
        ],
        out_specs=pl.BlockSpec((ROWS_BLK, DIM), lambda i: (i, 0)),
        out_shape=jax.ShapeDtypeStruct((N, DIM), jnp.float32),
    )(out_p, histd, b2d)


@jax.jit
def kernel(hypergraph, embedding, W, b):
    zeros = jnp.zeros((N, DIM), jnp.float32)

    xa = _tc_pre(embedding, W)
    e_p, histb = _sc_phase1(xa, hypergraph, zeros)
    ea = _tc_mid(e_p, histb)
    out_p, histd = _sc_phase2(ea, hypergraph, zeros)
    return _tc_final(out_p, histd, b.reshape(1, DIM))
